# baseline (speedup 1.0000x reference)
import functools

import jax
import jax.numpy as jnp
from jax import lax
from jax.experimental import pallas as pl
from jax.experimental.pallas import tpu as pltpu
from jax.experimental.pallas import tpu_sc as plsc

_L = 16


@functools.partial(jax.jit, static_argnums=(2, 3, 4, 5))
def _edge_sc(x3, idx2, BN, D, K, G):
    NC, NS = 2, 16
    NW = NC * NS
    NPW = BN // NW
    TR = 2 * G * K
    NCH = TR // 128
    n_groups = NPW // G
    NBUF = 2
    NJ = D // _L

    mesh = plsc.VectorSubcoreMesh(core_axis_name="c", subcore_axis_name="s")

    @functools.partial(
        pl.kernel,
        mesh=mesh,
        out_type=jax.ShapeDtypeStruct((BN * K * 2, 1, D), jnp.float32),
        scratch_types=[
            pltpu.VMEM((NBUF, NCH, 1, 128), jnp.int32),
            pltpu.VMEM((NBUF, TR, 1, D), jnp.float32),
            pltpu.SemaphoreType.DMA,
            pltpu.SemaphoreType.DMA,
            pltpu.SemaphoreType.DMA,
            pltpu.SemaphoreType.DMA,
        ],
    )
    def k(x_hbm, idx_hbm, out_hbm, idx_v, gat_v, g0, g1, s0, s1):
        gsem = (g0, g1)
        ssem = (s0, s1)
        wid = lax.axis_index("s") * NC + lax.axis_index("c")
        n0 = wid * NPW

        def issue_in(g, b):
            row0 = (n0 + g * G) * K * 2 // 128
            pltpu.sync_copy(idx_hbm.at[pl.ds(row0, NCH)], idx_v.at[b])
            for c in range(NCH):
                pltpu.async_copy(
                    x_hbm.at[idx_v.at[b, c, 0]],
                    gat_v.at[b, pl.ds(c * 128, 128)],
                    gsem[b])

        def wait_in(b):
            for c in range(NCH):
                pltpu.make_async_copy(
                    x_hbm.at[idx_v.at[b, c, 0]],
                    gat_v.at[b, pl.ds(c * 128, 128)],
                    gsem[b]).wait()

        def issue_out(g, b):
            r0 = (n0 + g * G) * K * 2
            pltpu.async_copy(
                gat_v.at[b], out_hbm.at[pl.ds(r0, TR)], ssem[b])

        def wait_out(b):
            pltpu.make_async_copy(
                gat_v.at[b], out_hbm.at[pl.ds(0, TR)], ssem[b]).wait()

        def compute(b):
            def i_body(i, car):
                ri = 2 * K * i
                cvecs = [gat_v[b, ri, 0, pl.ds(j * _L, _L)] for j in range(NJ)]

                def k_body(k4, car2):
                    for u in range(4):
                        r = ri + 2 * (k4 * 4 + u) + 1
                        for j in range(NJ):
                            sl = pl.ds(j * _L, _L)
                            gat_v[b, r, 0, sl] = gat_v[b, r, 0, sl] - cvecs[j]
                    return car2

                return lax.fori_loop(0, K // 4, k_body, car)

            lax.fori_loop(0, G, i_body, 0)

        issue_in(0, 0)

        def pair_body(gg, car):
            for b in range(NBUF):
                g2 = gg * NBUF + b
                nxt = 1 - b

                @pl.when(g2 >= 1)
                def _():
                    wait_out(nxt)

                @pl.when(g2 + 1 < n_groups)
                def _():
                    issue_in(g2 + 1, nxt)

                wait_in(b)
                compute(b)
                issue_out(g2, b)
            return car

        lax.fori_loop(0, n_groups // NBUF, pair_body, 0)
        wait_out((n_groups - 1) % NBUF)

    return k(x3, idx2)


def kernel(X_inputs, nn_idx):
    B, N, D = X_inputs.shape
    K = nn_idx.shape[-1]
    x3 = X_inputs.reshape(B * N, 1, D)
    offs = (jnp.arange(B, dtype=jnp.int32) * N).reshape(B, 1, 1)
    nbr_ids = nn_idx.astype(jnp.int32) + offs
    ctr_ids = jnp.broadcast_to(
        (jnp.arange(B * N, dtype=jnp.int32)).reshape(B, N, 1), (B, N, K))
    idx2 = jnp.stack([ctr_ids, nbr_ids], axis=-1).reshape(B * N * K * 2 // 128, 1, 128)
    out = _edge_sc(x3, idx2, B * N, D, K, 8)
    return out.reshape(B, N, K, 2 * D)
